# baseline (speedup 1.0000x reference)
import jax
import jax.numpy as jnp
from jax.experimental import pallas as pl

_NP = 16
_K = 16
_W_NORMAL = 1.0
_W_SURFVAR = 1.0


def _jacobi3(a, pp):
    one = jnp.ones((1, pp), jnp.float32)
    zero = jnp.zeros((1, pp), jnp.float32)
    v = [[one, zero, zero], [zero, one, zero], [zero, zero, one]]
    for _ in range(4):
        for (p, q) in ((0, 1), (0, 2), (1, 2)):
            r = 3 - p - q
            app = a[(p, p)]
            aqq = a[(q, q)]
            apq = a[(p, q)]
            theta = (aqq - app) * 0.5 / apq
            sgn = jnp.where(theta >= 0.0, 1.0, -1.0)
            t = sgn / (jnp.abs(theta) + jnp.sqrt(theta * theta + 1.0))
            t = jnp.where(apq == 0.0, 0.0, t)
            c = jax.lax.rsqrt(t * t + 1.0)
            s = t * c
            a[(p, p)] = app - t * apq
            a[(q, q)] = aqq + t * apq
            a[(p, q)] = zero
            rp = (min(r, p), max(r, p))
            rq = (min(r, q), max(r, q))
            arp = a[rp]
            arq = a[rq]
            a[rp] = c * arp - s * arq
            a[rq] = s * arp + c * arq
            for i in range(3):
                vip = v[i][p]
                viq = v[i][q]
                v[i][p] = c * vip - s * viq
                v[i][q] = s * vip + c * viq
    return [a[(0, 0)], a[(1, 1)], a[(2, 2)]], v


def _make_body(npat, pp):
    w_n = float(_W_NORMAL / (npat * pp * 3))
    w_s = float(_W_SURFVAR / (npat * pp))

    def body(x_ref, mom_ref, out_ref):
        i = pl.program_id(0)
        rows_i = jax.lax.broadcasted_iota(jnp.int32, (pp, pp), 0)
        cols_i = jax.lax.broadcasted_iota(jnp.int32, (pp, pp), 1)

        A0s = []
        for p in range(4):
            x = x_ref[p]
            xt = mom_ref[p, 6:9]
            D = jnp.zeros((pp, pp), jnp.float32)
            for c in range(3):
                dc = x[:, c:c + 1] - xt[c:c + 1, :]
                D = D + dc * dc
            A0s.append(jnp.where(rows_i == cols_i, jnp.inf, D))

        def sel_step(_, As):
            cmins = [jnp.min(Ap, axis=0, keepdims=True) for Ap in As]
            return tuple(jnp.where(Ap == cm, jnp.inf, Ap)
                         for Ap, cm in zip(As, cmins))

        As = jax.lax.fori_loop(0, _K - 1, sel_step, tuple(A0s))

        contrib = jnp.zeros((1, 1), jnp.float32)
        for p in range(4):
            contrib = contrib + _patch_tail(
                mom_ref[p], As[p], rows_i, cols_i, pp, w_n, w_s)

        @pl.when(i == 0)
        def _init():
            out_ref[:, :] = jnp.zeros((1, 1), jnp.float32)

        out_ref[:, :] = out_ref[:, :] + contrib

    return body


def _patch_tail(mom16, A, rows_i, cols_i, pp, w_n, w_s):
        xt = mom16[6:9]
        M = jnp.where(jnp.isinf(A) & (rows_i != cols_i), 1.0, 0.0)

        x0 = xt[0:1]
        x1 = xt[1:2]
        x2 = xt[2:3]
        F = jax.lax.dot(mom16[0:9], M, preferred_element_type=jnp.float32)
        k = float(_K - 1)
        s0 = F[6:7]
        s1 = F[7:8]
        s2 = F[8:9]
        a = {
            (0, 0): F[0:1] - 2.0 * x0 * s0 + k * x0 * x0,
            (1, 1): F[1:2] - 2.0 * x1 * s1 + k * x1 * x1,
            (2, 2): F[2:3] - 2.0 * x2 * s2 + k * x2 * x2,
            (0, 1): F[3:4] - x0 * s1 - x1 * s0 + k * x0 * x1,
            (0, 2): F[4:5] - x0 * s2 - x2 * s0 + k * x0 * x2,
            (1, 2): F[5:6] - x1 * s2 - x2 * s1 + k * x1 * x2,
        }
        tr = a[(0, 0)] + a[(1, 1)] + a[(2, 2)]
        w, v = _jacobi3(a, pp)
        w0, w1, w2 = w
        wmin = jnp.minimum(w0, jnp.minimum(w1, w2))
        surf_var = wmin / tr

        m0 = w0 == wmin
        m1 = w1 == wmin
        n = [jnp.where(m0, v[c][0], jnp.where(m1, v[c][1], v[c][2]))
             for c in range(3)]
        inv_norm = jax.lax.rsqrt(n[0] * n[0] + n[1] * n[1] + n[2] * n[2])
        n = [jnp.abs(nc * inv_norm) for nc in n]

        s_norm = jnp.zeros((1, 1), jnp.float32)
        for c in range(3):
            mean_c = jnp.sum(n[c], axis=1, keepdims=True) * (1.0 / pp)
            dev = n[c] - mean_c
            s_norm = s_norm + jnp.sum(dev * dev, axis=1, keepdims=True)
        s_sv = jnp.sum(surf_var, axis=1, keepdims=True)
        return s_norm * w_n + s_sv * w_s


def kernel(pointCloud):
    B, N, _ = pointCloud.shape
    npat = B * _NP
    pp = N // _NP
    x = pointCloud.reshape(npat, pp, 3).astype(jnp.float32)
    xt = jnp.swapaxes(x, 1, 2)
    c0 = xt[:, 0:1]
    c1 = xt[:, 1:2]
    c2 = xt[:, 2:3]
    mom = jnp.concatenate(
        [c0 * c0, c1 * c1, c2 * c2, c0 * c1, c0 * c2, c1 * c2, xt,
         jnp.zeros((npat, 7, pp), jnp.float32)], axis=1)

    out = pl.pallas_call(
        _make_body(npat, pp),
        grid=(npat // 4,),
        in_specs=[
            pl.BlockSpec((4, pp, 3), lambda i: (i, 0, 0)),
            pl.BlockSpec((4, 16, pp), lambda i: (i, 0, 0)),
        ],
        out_specs=pl.BlockSpec((1, 1), lambda i: (0, 0)),
        out_shape=jax.ShapeDtypeStruct((1, 1), jnp.float32),
    )(x, mom)
    return out[0, 0]


# speedup vs baseline: 1.0123x; 1.0123x over previous
import jax
import jax.numpy as jnp
from jax.experimental import pallas as pl

_NP = 16
_K = 16
_W_NORMAL = 1.0
_W_SURFVAR = 1.0


def _jacobi3(a, pp):
    one = jnp.ones((1, pp), jnp.float32)
    zero = jnp.zeros((1, pp), jnp.float32)
    v = [[one, zero, zero], [zero, one, zero], [zero, zero, one]]
    for _ in range(4):
        for (p, q) in ((0, 1), (0, 2), (1, 2)):
            r = 3 - p - q
            app = a[(p, p)]
            aqq = a[(q, q)]
            apq = a[(p, q)]
            theta = (aqq - app) * 0.5 / apq
            sgn = jnp.where(theta >= 0.0, 1.0, -1.0)
            t = sgn / (jnp.abs(theta) + jnp.sqrt(theta * theta + 1.0))
            t = jnp.where(apq == 0.0, 0.0, t)
            c = jax.lax.rsqrt(t * t + 1.0)
            s = t * c
            a[(p, p)] = app - t * apq
            a[(q, q)] = aqq + t * apq
            a[(p, q)] = zero
            rp = (min(r, p), max(r, p))
            rq = (min(r, q), max(r, q))
            arp = a[rp]
            arq = a[rq]
            a[rp] = c * arp - s * arq
            a[rq] = s * arp + c * arq
            for i in range(3):
                vip = v[i][p]
                viq = v[i][q]
                v[i][p] = c * vip - s * viq
                v[i][q] = s * vip + c * viq
    return [a[(0, 0)], a[(1, 1)], a[(2, 2)]], v


def _make_body(npat, pp):
    w_n = float(_W_NORMAL / (npat * pp * 3))
    w_s = float(_W_SURFVAR / (npat * pp))

    def body(x_ref, mom_ref, out_ref):
        i = pl.program_id(0)
        rows_i = jax.lax.broadcasted_iota(jnp.int32, (pp, pp), 0)
        cols_i = jax.lax.broadcasted_iota(jnp.int32, (pp, pp), 1)

        A0s = []
        for p in range(8):
            x = x_ref[p]
            xt = mom_ref[p, 6:9]
            D = jnp.zeros((pp, pp), jnp.float32)
            for c in range(3):
                dc = x[:, c:c + 1] - xt[c:c + 1, :]
                D = D + dc * dc
            A0s.append(jnp.where(rows_i == cols_i, jnp.inf, D))

        def sel_step(_, As):
            cmins = [jnp.min(Ap, axis=0, keepdims=True) for Ap in As]
            return tuple(jnp.where(Ap == cm, jnp.inf, Ap)
                         for Ap, cm in zip(As, cmins))

        As = jax.lax.fori_loop(0, _K - 1, sel_step, tuple(A0s))

        contrib = jnp.zeros((1, 1), jnp.float32)
        for p in range(8):
            contrib = contrib + _patch_tail(
                mom_ref[p], As[p], rows_i, cols_i, pp, w_n, w_s)

        @pl.when(i == 0)
        def _init():
            out_ref[:, :] = jnp.zeros((1, 1), jnp.float32)

        out_ref[:, :] = out_ref[:, :] + contrib

    return body


def _patch_tail(mom16, A, rows_i, cols_i, pp, w_n, w_s):
        xt = mom16[6:9]
        M = jnp.where(jnp.isinf(A) & (rows_i != cols_i), 1.0, 0.0)

        x0 = xt[0:1]
        x1 = xt[1:2]
        x2 = xt[2:3]
        F = jax.lax.dot(mom16[0:9], M, preferred_element_type=jnp.float32)
        k = float(_K - 1)
        s0 = F[6:7]
        s1 = F[7:8]
        s2 = F[8:9]
        a = {
            (0, 0): F[0:1] - 2.0 * x0 * s0 + k * x0 * x0,
            (1, 1): F[1:2] - 2.0 * x1 * s1 + k * x1 * x1,
            (2, 2): F[2:3] - 2.0 * x2 * s2 + k * x2 * x2,
            (0, 1): F[3:4] - x0 * s1 - x1 * s0 + k * x0 * x1,
            (0, 2): F[4:5] - x0 * s2 - x2 * s0 + k * x0 * x2,
            (1, 2): F[5:6] - x1 * s2 - x2 * s1 + k * x1 * x2,
        }
        tr = a[(0, 0)] + a[(1, 1)] + a[(2, 2)]
        w, v = _jacobi3(a, pp)
        w0, w1, w2 = w
        wmin = jnp.minimum(w0, jnp.minimum(w1, w2))
        surf_var = wmin / tr

        m0 = w0 == wmin
        m1 = w1 == wmin
        n = [jnp.where(m0, v[c][0], jnp.where(m1, v[c][1], v[c][2]))
             for c in range(3)]
        inv_norm = jax.lax.rsqrt(n[0] * n[0] + n[1] * n[1] + n[2] * n[2])
        n = [jnp.abs(nc * inv_norm) for nc in n]

        s_norm = jnp.zeros((1, 1), jnp.float32)
        for c in range(3):
            mean_c = jnp.sum(n[c], axis=1, keepdims=True) * (1.0 / pp)
            dev = n[c] - mean_c
            s_norm = s_norm + jnp.sum(dev * dev, axis=1, keepdims=True)
        s_sv = jnp.sum(surf_var, axis=1, keepdims=True)
        return s_norm * w_n + s_sv * w_s


def kernel(pointCloud):
    B, N, _ = pointCloud.shape
    npat = B * _NP
    pp = N // _NP
    x = pointCloud.reshape(npat, pp, 3).astype(jnp.float32)
    xt = jnp.swapaxes(x, 1, 2)
    c0 = xt[:, 0:1]
    c1 = xt[:, 1:2]
    c2 = xt[:, 2:3]
    mom = jnp.concatenate(
        [c0 * c0, c1 * c1, c2 * c2, c0 * c1, c0 * c2, c1 * c2, xt,
         jnp.zeros((npat, 7, pp), jnp.float32)], axis=1)

    out = pl.pallas_call(
        _make_body(npat, pp),
        grid=(npat // 8,),
        in_specs=[
            pl.BlockSpec((8, pp, 3), lambda i: (i, 0, 0)),
            pl.BlockSpec((8, 16, pp), lambda i: (i, 0, 0)),
        ],
        out_specs=pl.BlockSpec((1, 1), lambda i: (0, 0)),
        out_shape=jax.ShapeDtypeStruct((1, 1), jnp.float32),
    )(x, mom)
    return out[0, 0]
